# baseline (speedup 1.0000x reference)
import jax
import jax.numpy as jnp
from jax.experimental import pallas as pl
from jax.experimental.pallas import tpu as pltpu

_BLOCK = 1024


def _router_block(xa_ref, xb_ref, wt_ref, b_ref, out_ref):
    wt = wt_ref[...]
    bias = b_ref[...]
    for k, x_ref in enumerate((xa_ref, xb_ref)):
        xb16 = x_ref[...].astype(jnp.bfloat16)
        logits = jnp.dot(xb16, wt, preferred_element_type=jnp.float32)
        logits = logits + bias
        m = jnp.max(logits, axis=-1, keepdims=True)
        e = jnp.exp(logits - m)
        out_ref[pl.ds(k * _BLOCK, _BLOCK), :] = (
            e / jnp.sum(e, axis=-1, keepdims=True))


def kernel(x, W, b):
    n_tokens, d = x.shape
    n_experts = W.shape[0]
    grid = (n_tokens // (2 * _BLOCK),)
    return pl.pallas_call(
        _router_block,
        grid=grid,
        in_specs=[
            pl.BlockSpec((_BLOCK, d), lambda i: (2 * i, 0)),
            pl.BlockSpec((_BLOCK, d), lambda i: (2 * i + 1, 0)),
            pl.BlockSpec((d, n_experts), lambda i: (0, 0)),
            pl.BlockSpec((1, n_experts), lambda i: (0, 0)),
        ],
        out_specs=pl.BlockSpec((2 * _BLOCK, n_experts), lambda i: (i, 0)),
        out_shape=jax.ShapeDtypeStruct((n_tokens, n_experts), jnp.float32),
        compiler_params=pltpu.CompilerParams(
            dimension_semantics=("arbitrary",),
        ),
    )(x, x, W.T.astype(jnp.bfloat16), b.reshape(1, n_experts))
